# baseline (speedup 1.0000x reference)
import functools

import jax
import jax.numpy as jnp
from jax import lax
from jax.experimental import pallas as pl
from jax.experimental.pallas import tpu as pltpu
from jax.experimental.pallas import tpu_sc as plsc

_VOCAB = 30522
_HIDDEN = 768
_SEQ = 4096
_BATCH = 32
_EPS = 1e-12

_LANES = 16
_BV = 5120
_VPAD = 30720
_VCHUNK = _VPAD // 2
_NBLK = _VCHUNK // _BV


def _sc_hist_body(base, ids_hbm, zeros_hbm, out_hbm, idx_v, counts_v):
    wid = lax.axis_index("s") * 2 + lax.axis_index("c")
    pltpu.sync_copy(ids_hbm.at[wid], idx_v)
    pltpu.sync_copy(zeros_hbm, counts_v)
    ones = jnp.full((_LANES,), 1.0, jnp.float32)

    def body(i, carry):
        v = idx_v[pl.ds(i * _LANES, _LANES)] - base
        mask = (v >= 0) & (v < _VCHUNK)
        plsc.addupdate_scatter(counts_v, [v], ones, mask=mask)
        return carry

    lax.fori_loop(0, _SEQ // _LANES, body, 0)
    pltpu.sync_copy(counts_v, out_hbm.at[wid])


def _sc_histogram(base, ids_t, zeros):
    mesh = plsc.VectorSubcoreMesh(core_axis_name="c", subcore_axis_name="s")
    return pl.kernel(
        functools.partial(_sc_hist_body, base),
        out_type=jax.ShapeDtypeStruct((_BATCH, _VCHUNK), jnp.float32),
        mesh=mesh,
        scratch_types=[
            pltpu.VMEM((_SEQ,), jnp.int32),
            pltpu.VMEM((_VCHUNK,), jnp.float32),
        ],
        compiler_params=pltpu.CompilerParams(needs_layout_passes=False),
    )(ids_t, zeros)


def _tc_chunk_a_body(counts_ref, table_ref, out_ref):
    i = pl.program_id(0)
    x = table_ref[...]
    mean = jnp.mean(x, axis=1, keepdims=True)
    xc = x - mean
    var = jnp.mean(xc * xc, axis=1, keepdims=True)
    xhat = xc * lax.rsqrt(var + _EPS)
    prod = jnp.dot(counts_ref[...], xhat, preferred_element_type=jnp.float32)

    @pl.when(i == 0)
    def _():
        out_ref[...] = prod

    @pl.when(i > 0)
    def _():
        out_ref[...] += prod


def _tc_chunk_b_body(counts_ref, table_ref, acc_ref, w_ref, b_ref, out_ref):
    i = pl.program_id(0)
    x = table_ref[...]
    mean = jnp.mean(x, axis=1, keepdims=True)
    xc = x - mean
    var = jnp.mean(xc * xc, axis=1, keepdims=True)
    xhat = xc * lax.rsqrt(var + _EPS)
    rows = _VCHUNK + i * _BV + lax.broadcasted_iota(jnp.int32, (_BV, 1), 0)
    xhat = jnp.where(rows < _VOCAB, xhat, 0.0)
    prod = jnp.dot(counts_ref[...], xhat, preferred_element_type=jnp.float32)

    @pl.when(i == 0)
    def _():
        out_ref[...] = acc_ref[...] + prod

    @pl.when(i > 0)
    def _():
        out_ref[...] += prod

    @pl.when(i == _NBLK - 1)
    def _():
        out_ref[...] = out_ref[...] * (w_ref[...] * (1.0 / _SEQ)) + b_ref[...]


def _tc_chunk_a(counts0, table):
    return pl.pallas_call(
        _tc_chunk_a_body,
        grid=(_NBLK,),
        in_specs=[
            pl.BlockSpec((_BATCH, _BV), lambda i: (0, i)),
            pl.BlockSpec((_BV, _HIDDEN), lambda i: (i, 0)),
        ],
        out_specs=pl.BlockSpec((_BATCH, _HIDDEN), lambda i: (0, 0)),
        out_shape=jax.ShapeDtypeStruct((_BATCH, _HIDDEN), jnp.float32),
    )(counts0, table)


def _tc_chunk_b(counts1, table, acc, w2, b2):
    return pl.pallas_call(
        _tc_chunk_b_body,
        grid=(_NBLK,),
        in_specs=[
            pl.BlockSpec((_BATCH, _BV), lambda i: (0, i)),
            pl.BlockSpec((_BV, _HIDDEN), lambda i: (i + _NBLK, 0)),
            pl.BlockSpec((_BATCH, _HIDDEN), lambda i: (0, 0)),
            pl.BlockSpec((1, _HIDDEN), lambda i: (0, 0)),
            pl.BlockSpec((1, _HIDDEN), lambda i: (0, 0)),
        ],
        out_specs=pl.BlockSpec((_BATCH, _HIDDEN), lambda i: (0, 0)),
        out_shape=jax.ShapeDtypeStruct((_BATCH, _HIDDEN), jnp.float32),
    )(counts1, table, acc, w2, b2)


def kernel(ids, word_embeddings, ln_weight, ln_bias):
    ids_t = ids.T.astype(jnp.int32)
    zeros = jnp.zeros((_VCHUNK,), jnp.float32)
    counts0 = _sc_histogram(0, ids_t, zeros)
    counts1 = _sc_histogram(_VCHUNK, ids_t, zeros)
    acc = _tc_chunk_a(counts0, word_embeddings)
    return _tc_chunk_b(
        counts1,
        word_embeddings,
        acc,
        ln_weight.reshape(1, _HIDDEN),
        ln_bias.reshape(1, _HIDDEN),
    )


# speedup vs baseline: 1.1035x; 1.1035x over previous
import jax
import jax.numpy as jnp
from jax import lax
from jax.experimental import pallas as pl
from jax.experimental.pallas import tpu as pltpu
from jax.experimental.pallas import tpu_sc as plsc

_VOCAB = 30522
_HIDDEN = 768
_SEQ = 4096
_BATCH = 32
_EPS = 1e-12

_LANES = 16
_BV = 6144
_VPAD = 30720
_NBLK = _VPAD // _BV


def _sc_hist_body(ids_hbm, zeros_hbm, out_hbm, idx_v, counts_v):
    wid = lax.axis_index("s") * 2 + lax.axis_index("c")
    pltpu.sync_copy(ids_hbm.at[wid], idx_v)
    pltpu.sync_copy(zeros_hbm, counts_v)
    ones = jnp.full((_LANES,), 1.0, jnp.float32)

    def body(i, carry):
        v = idx_v[pl.ds(i * _LANES, _LANES)]
        plsc.addupdate_scatter(counts_v, [v], ones)
        return carry

    lax.fori_loop(0, _SEQ // _LANES, body, 0)
    pltpu.sync_copy(counts_v, out_hbm.at[wid])


def _sc_histogram(ids_t, zeros):
    mesh = plsc.VectorSubcoreMesh(core_axis_name="c", subcore_axis_name="s")
    return pl.kernel(
        _sc_hist_body,
        out_type=jax.ShapeDtypeStruct((_BATCH, _VPAD), jnp.float32),
        mesh=mesh,
        scratch_types=[
            pltpu.VMEM((_SEQ,), jnp.int32),
            pltpu.VMEM((_VPAD,), jnp.float32),
        ],
        compiler_params=pltpu.CompilerParams(needs_layout_passes=False),
    )(ids_t, zeros)


def _tc_body(counts_ref, table_ref, w_ref, b_ref, out_ref, acc_ref):
    i = pl.program_id(0)
    x = table_ref[...]
    mean = jnp.mean(x, axis=1, keepdims=True)
    xc = x - mean
    var = jnp.mean(xc * xc, axis=1, keepdims=True)
    xhat = xc * lax.rsqrt(var + _EPS)
    rows = i * _BV + lax.broadcasted_iota(jnp.int32, (_BV, 1), 0)
    xhat = jnp.where(rows < _VOCAB, xhat, 0.0)
    prod = jnp.dot(counts_ref[...], xhat, preferred_element_type=jnp.float32)

    @pl.when(i == 0)
    def _():
        acc_ref[...] = prod

    @pl.when(i > 0)
    def _():
        acc_ref[...] += prod

    @pl.when(i == _NBLK - 1)
    def _():
        out_ref[...] = acc_ref[...] * (w_ref[...] * (1.0 / _SEQ)) + b_ref[...]


def _tc_reduce(counts_t, table, w2, b2):
    return pl.pallas_call(
        _tc_body,
        grid=(_NBLK,),
        in_specs=[
            pl.BlockSpec((_BATCH, _BV), lambda i: (0, i)),
            pl.BlockSpec((_BV, _HIDDEN), lambda i: (i, 0)),
            pl.BlockSpec((1, _HIDDEN), lambda i: (0, 0)),
            pl.BlockSpec((1, _HIDDEN), lambda i: (0, 0)),
        ],
        out_specs=pl.BlockSpec((_BATCH, _HIDDEN), lambda i: (0, 0)),
        out_shape=jax.ShapeDtypeStruct((_BATCH, _HIDDEN), jnp.float32),
        scratch_shapes=[pltpu.VMEM((_BATCH, _HIDDEN), jnp.float32)],
    )(counts_t, table, w2, b2)


def kernel(ids, word_embeddings, ln_weight, ln_bias):
    ids_t = ids.T.astype(jnp.int32)
    zeros = jnp.zeros((_VPAD,), jnp.float32)
    counts_t = _sc_histogram(ids_t, zeros)
    return _tc_reduce(
        counts_t,
        word_embeddings,
        ln_weight.reshape(1, _HIDDEN),
        ln_bias.reshape(1, _HIDDEN),
    )


# speedup vs baseline: 1.1255x; 1.0200x over previous
import jax
import jax.numpy as jnp
from jax import lax
from jax.experimental import pallas as pl
from jax.experimental.pallas import tpu as pltpu
from jax.experimental.pallas import tpu_sc as plsc

_VOCAB = 30522
_HIDDEN = 768
_SEQ = 4096
_BATCH = 32
_EPS = 1e-12

_LANES = 16
_BV = 5120
_VPAD = 30720
_NBLK = _VPAD // _BV


_UNROLL = 4


def _sc_hist_body(ids_hbm, zeros_hbm, out_hbm, idx_v, counts_v, sem_i, sem_z):
    wid = lax.axis_index("s") * 2 + lax.axis_index("c")
    cp_i = pltpu.async_copy(ids_hbm.at[wid], idx_v, sem_i)
    cp_z = pltpu.async_copy(zeros_hbm, counts_v, sem_z)
    cp_i.wait()
    cp_z.wait()
    ones = jnp.full((_LANES,), 1.0, jnp.float32)

    def body(i, carry):
        for k in range(_UNROLL):
            v = idx_v[pl.ds(i * (_LANES * _UNROLL) + k * _LANES, _LANES)]
            plsc.addupdate_scatter(counts_v, [v], ones)
        return carry

    lax.fori_loop(0, _SEQ // (_LANES * _UNROLL), body, 0)
    pltpu.sync_copy(counts_v, out_hbm.at[wid])


def _sc_histogram(ids_t, zeros):
    mesh = plsc.VectorSubcoreMesh(core_axis_name="c", subcore_axis_name="s")
    return pl.kernel(
        _sc_hist_body,
        out_type=jax.ShapeDtypeStruct((_BATCH, _VPAD), jnp.float32),
        mesh=mesh,
        scratch_types=[
            pltpu.VMEM((_SEQ,), jnp.int32),
            pltpu.VMEM((_VPAD,), jnp.float32),
            pltpu.SemaphoreType.DMA,
            pltpu.SemaphoreType.DMA,
        ],
        compiler_params=pltpu.CompilerParams(needs_layout_passes=False),
    )(ids_t, zeros)


def _tc_body(counts_ref, table_ref, w_ref, b_ref, out_ref, acc_ref):
    i = pl.program_id(0)
    x = table_ref[...]
    mean = jnp.mean(x, axis=1, keepdims=True)
    xc = x - mean
    var = jnp.mean(xc * xc, axis=1, keepdims=True)
    xhat = xc * lax.rsqrt(var + _EPS)
    rows = i * _BV + lax.broadcasted_iota(jnp.int32, (_BV, 1), 0)
    xhat = jnp.where(rows < _VOCAB, xhat, 0.0)
    prod = jnp.dot(counts_ref[...], xhat, preferred_element_type=jnp.float32)

    @pl.when(i == 0)
    def _():
        acc_ref[...] = prod

    @pl.when(i > 0)
    def _():
        acc_ref[...] += prod

    @pl.when(i == _NBLK - 1)
    def _():
        out_ref[...] = acc_ref[...] * (w_ref[...] * (1.0 / _SEQ)) + b_ref[...]


def _tc_reduce(counts_t, table, w2, b2):
    return pl.pallas_call(
        _tc_body,
        grid=(_NBLK,),
        in_specs=[
            pl.BlockSpec((_BATCH, _BV), lambda i: (0, i)),
            pl.BlockSpec((_BV, _HIDDEN), lambda i: (i, 0)),
            pl.BlockSpec((1, _HIDDEN), lambda i: (0, 0)),
            pl.BlockSpec((1, _HIDDEN), lambda i: (0, 0)),
        ],
        out_specs=pl.BlockSpec((_BATCH, _HIDDEN), lambda i: (0, 0)),
        out_shape=jax.ShapeDtypeStruct((_BATCH, _HIDDEN), jnp.float32),
        scratch_shapes=[pltpu.VMEM((_BATCH, _HIDDEN), jnp.float32)],
    )(counts_t, table, w2, b2)


def kernel(ids, word_embeddings, ln_weight, ln_bias):
    ids_t = ids.T.astype(jnp.int32)
    zeros = jnp.zeros((_VPAD,), jnp.float32)
    counts_t = _sc_histogram(ids_t, zeros)
    return _tc_reduce(
        counts_t,
        word_embeddings,
        ln_weight.reshape(1, _HIDDEN),
        ln_bias.reshape(1, _HIDDEN),
    )
